# baseline (speedup 1.0000x reference)
import functools

import jax
import jax.numpy as jnp
from jax import lax
from jax.experimental import pallas as pl
from jax.experimental.pallas import tpu as pltpu
from jax.experimental.pallas import tpu_sc as plsc

N = 100000
D = 128
C = 400
NC = 2
NS = 16
NW = NC * NS
NUM_CHUNKS = N // C
KMAX = (NUM_CHUNKS + NW - 1) // NW
RG = 16


def _body(xf_hbm, wp_hbm, out_hbm,
          xin_a, xin_b, wt_v, out_a, out_b,
          sem_ia, sem_ib, sem_oa, sem_ob):
  cid = lax.axis_index("c")
  sid = lax.axis_index("s")
  wid = sid * NC + cid

  pltpu.sync_copy(wp_hbm, wt_v)

  bufs = ((xin_a, out_a, sem_ia, sem_oa),
          (xin_b, out_b, sem_ib, sem_ob))
  lane15 = lax.iota(jnp.int32, 16) == (16 - 1)
  w0 = [wt_v[0, pl.ds(c * 16, 16)] for c in range(D // 16)]
  w1 = [wt_v[1, pl.ds(c * 16, 16)] for c in range(D // 16)]

  def start_in(k):
    xin, _, si, _ = bufs[k % 2]
    base = (k * NW + wid) * C
    pltpu.async_copy(xf_hbm.at[pl.ds(2 * base, 2 * C)], xin, si)

  def wait_in(k):
    xin, _, si, _ = bufs[k % 2]
    pltpu.make_async_copy(xf_hbm.at[pl.ds(0, 2 * C)], xin, si).wait()

  def wait_out(p):
    _, o, _, so = bufs[p]
    pltpu.make_async_copy(o, out_hbm.at[pl.ds(0, C), :], so).wait()

  def fill_and_send(k):
    xin, o, _, so = bufs[k % 2]
    base = (k * NW + wid) * C

    def row_group(j, carry):
      r0 = j * RG
      lanes = [xin[pl.ds(2 * r0 + 16 * h, 16)] for h in range(2)]
      for u in range(RG):
        r = r0 + u
        src = lanes[u // 8]
        m = jnp.full((16,), src[(2 * u) % 16]) != 0.0
        for c in range(D // 16 - 1):
          o[r, pl.ds(c * 16, 16)] = jnp.where(m, w1[c], w0[c])
        tail = jnp.where(m, w1[D // 16 - 1], w0[D // 16 - 1])
        xb = jnp.full((16,), src[(2 * u + 1) % 16], jnp.float32)
        o[r, pl.ds(D - 16, 16)] = jnp.where(lane15, xb, tail)
      return carry

    lax.fori_loop(0, C // RG, row_group, 0)
    pltpu.async_copy(o, out_hbm.at[pl.ds(base, C), :], so)

  start_in(0)
  for k in range(KMAX):
    def step(k=k):
      if k + 1 < KMAX:
        if (k + 1) * NW + NW - 1 < NUM_CHUNKS:
          start_in(k + 1)
        else:
          pl.when((k + 1) * NW + wid < NUM_CHUNKS)(lambda: start_in(k + 1))
      wait_in(k)
      if k >= 2:
        wait_out(k % 2)
      fill_and_send(k)
    if k * NW + NW - 1 < NUM_CHUNKS:
      step()
    else:
      pl.when(k * NW + wid < NUM_CHUNKS)(step)
  wait_out(0)
  wait_out(1)


@jax.jit
def kernel(x, W):
  xf = x.reshape(2 * N)
  wp = jnp.pad(W, ((0, 0), (0, 1)))
  mesh = plsc.VectorSubcoreMesh(
      core_axis_name="c", subcore_axis_name="s", num_cores=NC,
      num_subcores=NS)
  run = pl.kernel(
      _body,
      out_type=jax.ShapeDtypeStruct((N, D), jnp.float32),
      mesh=mesh,
      compiler_params=pltpu.CompilerParams(
          use_tc_tiling_on_sc=False, needs_layout_passes=False),
      scratch_types=[
          pltpu.VMEM((2 * C,), jnp.float32),
          pltpu.VMEM((2 * C,), jnp.float32),
          pltpu.VMEM((2, D), jnp.float32),
          pltpu.VMEM((C, D), jnp.float32),
          pltpu.VMEM((C, D), jnp.float32),
          pltpu.SemaphoreType.DMA,
          pltpu.SemaphoreType.DMA,
          pltpu.SemaphoreType.DMA,
          pltpu.SemaphoreType.DMA,
      ],
  )
  return run(xf, wp)


# speedup vs baseline: 1.0835x; 1.0835x over previous
import functools

import jax
import jax.numpy as jnp
from jax import lax
from jax.experimental import pallas as pl
from jax.experimental.pallas import tpu as pltpu
from jax.experimental.pallas import tpu_sc as plsc

N = 100000
D = 128
C = 400
NC = 2
NS = 16
NW = NC * NS
NUM_CHUNKS = N // C
KMAX = (NUM_CHUNKS + NW - 1) // NW
RG = 8


def _body(xf_hbm, wp_hbm, out_hbm,
          xin_a, xin_b, wt_v, out_a, out_b,
          sem_ia, sem_ib, sem_oa, sem_ob):
  cid = lax.axis_index("c")
  sid = lax.axis_index("s")
  wid = sid * NC + cid

  pltpu.sync_copy(wp_hbm, wt_v)

  bufs = ((xin_a, out_a, sem_ia, sem_oa),
          (xin_b, out_b, sem_ib, sem_ob))
  lane15 = lax.iota(jnp.int32, 16) == (16 - 1)
  w0 = [wt_v[0, pl.ds(c * 16, 16)] for c in range(D // 16)]
  w1 = [wt_v[1, pl.ds(c * 16, 16)] for c in range(D // 16)]

  def start_in(k):
    xin, _, si, _ = bufs[k % 2]
    base = (k * NW + wid) * C
    pltpu.async_copy(xf_hbm.at[pl.ds(2 * base, 2 * C)], xin, si)

  def wait_in(k):
    xin, _, si, _ = bufs[k % 2]
    pltpu.make_async_copy(xf_hbm.at[pl.ds(0, 2 * C)], xin, si).wait()

  def wait_out(p):
    _, o, _, so = bufs[p]
    pltpu.make_async_copy(o, out_hbm.at[pl.ds(0, C), :], so).wait()

  def fill_and_send(k):
    xin, o, _, so = bufs[k % 2]
    base = (k * NW + wid) * C

    @functools.partial(plsc.parallel_loop, 0, C // RG)
    def row_group(j):
      r0 = j * RG
      src = xin[pl.ds(2 * r0, 16)]
      for u in range(RG):
        r = r0 + u
        m = jnp.full((16,), src[2 * u]) != 0.0
        for c in range(D // 16 - 1):
          o[r, pl.ds(c * 16, 16)] = jnp.where(m, w1[c], w0[c])
        tail = jnp.where(m, w1[D // 16 - 1], w0[D // 16 - 1])
        xb = jnp.full((16,), src[2 * u + 1], jnp.float32)
        o[r, pl.ds(D - 16, 16)] = jnp.where(lane15, xb, tail)

    pltpu.async_copy(o, out_hbm.at[pl.ds(base, C), :], so)

  start_in(0)
  for k in range(KMAX):
    def step(k=k):
      if k + 1 < KMAX:
        if (k + 1) * NW + NW - 1 < NUM_CHUNKS:
          start_in(k + 1)
        else:
          pl.when((k + 1) * NW + wid < NUM_CHUNKS)(lambda: start_in(k + 1))
      wait_in(k)
      if k >= 2:
        wait_out(k % 2)
      fill_and_send(k)
    if k * NW + NW - 1 < NUM_CHUNKS:
      step()
    else:
      pl.when(k * NW + wid < NUM_CHUNKS)(step)
  wait_out(0)
  wait_out(1)


@jax.jit
def kernel(x, W):
  xf = x.reshape(2 * N)
  wp = jnp.pad(W, ((0, 0), (0, 1)))
  mesh = plsc.VectorSubcoreMesh(
      core_axis_name="c", subcore_axis_name="s", num_cores=NC,
      num_subcores=NS)
  run = pl.kernel(
      _body,
      out_type=jax.ShapeDtypeStruct((N, D), jnp.float32),
      mesh=mesh,
      compiler_params=pltpu.CompilerParams(
          use_tc_tiling_on_sc=False, needs_layout_passes=False),
      scratch_types=[
          pltpu.VMEM((2 * C,), jnp.float32),
          pltpu.VMEM((2 * C,), jnp.float32),
          pltpu.VMEM((2, D), jnp.float32),
          pltpu.VMEM((C, D), jnp.float32),
          pltpu.VMEM((C, D), jnp.float32),
          pltpu.SemaphoreType.DMA,
          pltpu.SemaphoreType.DMA,
          pltpu.SemaphoreType.DMA,
          pltpu.SemaphoreType.DMA,
      ],
  )
  return run(xf, wp)


# speedup vs baseline: 2.5429x; 2.3469x over previous
import functools

import jax
import jax.numpy as jnp
from jax import lax
from jax.experimental import pallas as pl
from jax.experimental.pallas import tpu as pltpu
from jax.experimental.pallas import tpu_sc as plsc

N = 100000
D = 128
C = 400
NC = 2
NS = 16
NW = NC * NS
NUM_CHUNKS = N // C
KMAX = (NUM_CHUNKS + NW - 1) // NW
RG = 16


def _body(x0_hbm, x1_hbm, wp_hbm, out_hbm,
          flg_a, flg_b, x1_a, x1_b, wt_v, out_a, out_b,
          sem_ia, sem_ib, sem_oa, sem_ob):
  cid = lax.axis_index("c")
  sid = lax.axis_index("s")
  wid = sid * NC + cid

  pltpu.sync_copy(wp_hbm, wt_v)

  bufs = ((flg_a, x1_a, out_a, sem_ia, sem_oa),
          (flg_b, x1_b, out_b, sem_ib, sem_ob))
  lane15 = lax.iota(jnp.int32, 16) == (16 - 1)
  w0 = [wt_v[0, pl.ds(c * 16, 16)] for c in range(D // 16)]
  w1 = [wt_v[1, pl.ds(c * 16, 16)] for c in range(D // 16)]

  def start_in(k):
    f, x, _, si, _ = bufs[k % 2]
    base = (k * NW + wid) * C
    pltpu.async_copy(x0_hbm.at[pl.ds(base, C)], f, si)
    pltpu.async_copy(x1_hbm.at[pl.ds(base, C)], x, si)

  def wait_in(k):
    f, x, _, si, _ = bufs[k % 2]
    pltpu.make_async_copy(x0_hbm.at[pl.ds(0, C)], f, si).wait()
    pltpu.make_async_copy(x1_hbm.at[pl.ds(0, C)], x, si).wait()

  def wait_out(p):
    _, _, o, _, so = bufs[p]
    pltpu.make_async_copy(o, out_hbm.at[pl.ds(0, C), :], so).wait()

  def fill_and_send(k):
    f, x, o, _, so = bufs[k % 2]
    base = (k * NW + wid) * C

    @functools.partial(plsc.parallel_loop, 0, C // RG)
    def row_group(j):
      r0 = j * RG
      fvec = f[pl.ds(r0, RG)]
      xvec = x[pl.ds(r0, RG)]
      for u in range(RG):
        r = r0 + u
        m = jnp.full((16,), fvec[u]) != 0.0
        for c in range(D // 16 - 1):
          o[r, pl.ds(c * 16, 16)] = jnp.where(m, w1[c], w0[c])
        tail = jnp.where(m, w1[D // 16 - 1], w0[D // 16 - 1])
        xb = jnp.full((16,), xvec[u], jnp.float32)
        o[r, pl.ds(D - 16, 16)] = jnp.where(lane15, xb, tail)

    pltpu.async_copy(o, out_hbm.at[pl.ds(base, C), :], so)

  start_in(0)
  for k in range(KMAX):
    def step(k=k):
      if k + 1 < KMAX:
        if (k + 1) * NW + NW - 1 < NUM_CHUNKS:
          start_in(k + 1)
        else:
          pl.when((k + 1) * NW + wid < NUM_CHUNKS)(lambda: start_in(k + 1))
      wait_in(k)
      if k >= 2:
        wait_out(k % 2)
      fill_and_send(k)
    if k * NW + NW - 1 < NUM_CHUNKS:
      step()
    else:
      pl.when(k * NW + wid < NUM_CHUNKS)(step)
  wait_out(0)
  wait_out(1)


@jax.jit
def kernel(x, W):
  xt = x.T
  x0 = xt[0]
  x1 = xt[1]
  wp = jnp.pad(W, ((0, 0), (0, 1)))
  mesh = plsc.VectorSubcoreMesh(
      core_axis_name="c", subcore_axis_name="s", num_cores=NC,
      num_subcores=NS)
  run = pl.kernel(
      _body,
      out_type=jax.ShapeDtypeStruct((N, D), jnp.float32),
      mesh=mesh,
      compiler_params=pltpu.CompilerParams(
          use_tc_tiling_on_sc=False, needs_layout_passes=False),
      scratch_types=[
          pltpu.VMEM((C,), jnp.float32),
          pltpu.VMEM((C,), jnp.float32),
          pltpu.VMEM((C,), jnp.float32),
          pltpu.VMEM((C,), jnp.float32),
          pltpu.VMEM((2, D), jnp.float32),
          pltpu.VMEM((C, D), jnp.float32),
          pltpu.VMEM((C, D), jnp.float32),
          pltpu.SemaphoreType.DMA,
          pltpu.SemaphoreType.DMA,
          pltpu.SemaphoreType.DMA,
          pltpu.SemaphoreType.DMA,
      ],
  )
  return run(x0, x1, wp)
